# baseline (speedup 1.0000x reference)
import functools

import jax
import jax.numpy as jnp
from jax import lax
from jax.experimental import pallas as pl
from jax.experimental.pallas import tpu as pltpu
from jax.experimental.pallas import tpu_sc as plsc

NC = 2
NS = 16
LANES = 16

B = 1024
L = 512
D = 128
BL = B * L
NW = NC * NS
PER_W = BL // NW
K = 128
CHUNKS = PER_W // K
CPS = L // K
GROUPS = D // LANES


def _body(seq_hbm, lab_hbm, tok_hbm, pos_hbm, seg_hbm, out_hbm,
          pos_v, tok_buf, seg_buf, idx_v, lab_v, sem_t, sem_s):
    wid = lax.axis_index("s") * NC + lax.axis_index("c")
    w_base = wid * PER_W

    pltpu.sync_copy(pos_hbm, pos_v)

    def chunk_body(c, _):
        base = w_base + c * K
        pltpu.sync_copy(seq_hbm.at[pl.ds(base, K)], idx_v)
        pltpu.sync_copy(lab_hbm.at[pl.ds(base, K)], lab_v)
        t_cp = pltpu.async_copy(tok_hbm.at[idx_v], tok_buf, sem_t)
        s_cp = pltpu.async_copy(seg_hbm.at[lab_v], seg_buf, sem_s)
        t_cp.wait()
        s_cp.wait()

        pos_off = (c % CPS) * K

        def tok_body(t, carry):
            for g in range(GROUPS):
                sl = pl.ds(g * LANES, LANES)
                tok_buf[t, sl] = (tok_buf[t, sl]
                                  + pos_v[pos_off + t, sl]
                                  + seg_buf[t, sl])
            return carry

        lax.fori_loop(0, K, tok_body, 0)
        pltpu.sync_copy(tok_buf, out_hbm.at[pl.ds(base, K)])
        return _

    lax.fori_loop(0, CHUNKS, chunk_body, 0)


@jax.jit
def _embed(seq_flat, lab_flat, token_table, position_table, segment_table):
    mesh = plsc.VectorSubcoreMesh(core_axis_name="c", subcore_axis_name="s")
    kfn = pl.kernel(
        _body,
        out_type=jax.ShapeDtypeStruct((BL, D), jnp.float32),
        mesh=mesh,
        scratch_types=[
            pltpu.VMEM((L, D), jnp.float32),
            pltpu.VMEM((K, D), jnp.float32),
            pltpu.VMEM((K, D), jnp.float32),
            pltpu.VMEM((K,), jnp.int32),
            pltpu.VMEM((K,), jnp.int32),
            pltpu.SemaphoreType.DMA,
            pltpu.SemaphoreType.DMA,
        ],
    )
    return kfn(seq_flat, lab_flat, token_table, position_table, segment_table)


def kernel(sequence, segment_label, token_table, position_table, segment_table):
    seq_flat = sequence.reshape(BL).astype(jnp.int32)
    lab_flat = segment_label.reshape(BL).astype(jnp.int32)
    out = _embed(seq_flat, lab_flat, token_table, position_table,
                 segment_table)
    return out.reshape(B, L, D)
